# baseline (speedup 1.0000x reference)
import functools
import jax
import jax.numpy as jnp
from jax import lax
from jax.experimental import pallas as pl
from jax.experimental.pallas import tpu as pltpu
from jax.experimental.pallas import tpu_sc as plsc

D = 1024
NB = 8
NC = 2
NS = 16
NW = NC * NS
R = 8
L = 16


def kernel(x, blocks, W):
    B, T, d = x.shape
    N = B * T
    rows_per_w = N // NW
    n_chunks = rows_per_w // R
    x2 = x.reshape(N, d)
    idx = blocks.reshape(N).astype(jnp.int32)
    mesh = plsc.VectorSubcoreMesh(core_axis_name="c", subcore_axis_name="s")

    @functools.partial(
        pl.kernel,
        mesh=mesh,
        out_type=jax.ShapeDtypeStruct((N, d), jnp.float32),
        scratch_types=[
            pltpu.VMEM((NB, d), jnp.float32),
            pltpu.VMEM((rows_per_w + L,), jnp.int32),
            pltpu.VMEM((R, d), jnp.float32),
            pltpu.VMEM((R, d), jnp.float32),
            pltpu.VMEM((R, d), jnp.float32),
            pltpu.VMEM((R, d), jnp.float32),
            pltpu.SemaphoreType.DMA,
            pltpu.SemaphoreType.DMA,
            pltpu.SemaphoreType.DMA,
            pltpu.SemaphoreType.DMA,
            pltpu.SemaphoreType.DMA,
            pltpu.SemaphoreType.DMA,
            pltpu.SemaphoreType.DMA,
            pltpu.SemaphoreType.DMA,
        ],
    )
    def sc_add(x_hbm, idx_hbm, w_hbm, out_hbm, wbuf, idxbuf,
               xb0, xb1, xb2, xb3, si0, si1, si2, si3, so0, so1, so2, so3):
        cid = lax.axis_index("c")
        sid = lax.axis_index("s")
        wid = sid * NC + cid
        base = wid * rows_per_w
        pltpu.sync_copy(w_hbm, wbuf)
        pltpu.sync_copy(idx_hbm.at[pl.ds(base, rows_per_w)],
                        idxbuf.at[pl.ds(0, rows_per_w)])

        bufs = (xb0, xb1, xb2, xb3)
        ins = (si0, si1, si2, si3)
        outs = (so0, so1, so2, so3)

        def row_chunk(xb, goff):
            bvec = idxbuf[pl.ds(goff, L)]
            for j in range(R // 4):
                bis = [bvec[j + m * (R // 4)] for m in range(4)]

                @plsc.parallel_loop(0, d // L, unroll=8)
                def col_body(k, bis=bis, j=j, xb=xb):
                    for m in range(4):
                        v = wbuf[bis[m], pl.ds(k * L, L)]
                        plsc.addupdate(xb.at[j + m * (R // 4), pl.ds(k * L, L)], v)

        pltpu.async_copy(x_hbm.at[pl.ds(base, R)], bufs[0], ins[0])

        def outer(g4, carry):
            for b in range(4):
                g = g4 * 4 + b
                nb = (b + 1) % 4
                @pl.when(g + 1 < n_chunks)
                def _():
                    @pl.when(g >= 3)
                    def _():
                        pltpu.make_async_copy(
                            bufs[nb], out_hbm.at[pl.ds(base + (g - 3) * R, R)],
                            outs[nb]).wait()

                    pltpu.async_copy(
                        x_hbm.at[pl.ds(base + (g + 1) * R, R)], bufs[nb],
                        ins[nb])

                pltpu.make_async_copy(
                    x_hbm.at[pl.ds(base + g * R, R)], bufs[b], ins[b]).wait()
                row_chunk(bufs[b], g * R)
                pltpu.async_copy(
                    bufs[b], out_hbm.at[pl.ds(base + g * R, R)], outs[b])
            return carry

        lax.fori_loop(0, n_chunks // 4, outer, 0)
        for b in range(4):
            g = n_chunks - 4 + b
            pltpu.make_async_copy(
                bufs[g % 4], out_hbm.at[pl.ds(base + g * R, R)],
                outs[g % 4]).wait()

    out = sc_add(x2, idx, W)
    return out.reshape(B, T, d)


# speedup vs baseline: 1.0982x; 1.0982x over previous
import functools
import jax
import jax.numpy as jnp
from jax import lax
from jax.experimental import pallas as pl
from jax.experimental.pallas import tpu as pltpu
from jax.experimental.pallas import tpu_sc as plsc

D = 1024
NB = 8
NC = 2
NS = 16
NW = NC * NS
R = 16
L = 16


def kernel(x, blocks, W):
    B, T, d = x.shape
    N = B * T
    rows_per_w = N // NW
    n_chunks = rows_per_w // R
    x2 = x.reshape(N, d)
    idx = blocks.reshape(N).astype(jnp.int32)
    mesh = plsc.VectorSubcoreMesh(core_axis_name="c", subcore_axis_name="s")

    @functools.partial(
        pl.kernel,
        mesh=mesh,
        out_type=jax.ShapeDtypeStruct((N, d), jnp.float32),
        scratch_types=[
            pltpu.VMEM((NB, d), jnp.float32),
            pltpu.VMEM((rows_per_w,), jnp.int32),
            pltpu.VMEM((R, d), jnp.float32),
            pltpu.VMEM((R, d), jnp.float32),
            pltpu.VMEM((R, d), jnp.float32),
            pltpu.VMEM((R, d), jnp.float32),
            pltpu.SemaphoreType.DMA,
            pltpu.SemaphoreType.DMA,
            pltpu.SemaphoreType.DMA,
            pltpu.SemaphoreType.DMA,
            pltpu.SemaphoreType.DMA,
            pltpu.SemaphoreType.DMA,
            pltpu.SemaphoreType.DMA,
            pltpu.SemaphoreType.DMA,
        ],
    )
    def sc_add(x_hbm, idx_hbm, w_hbm, out_hbm, wbuf, idxbuf,
               xb0, xb1, xb2, xb3, si0, si1, si2, si3, so0, so1, so2, so3):
        cid = lax.axis_index("c")
        sid = lax.axis_index("s")
        wid = sid * NC + cid
        base = wid * rows_per_w
        pltpu.sync_copy(w_hbm, wbuf)
        pltpu.sync_copy(idx_hbm.at[pl.ds(base, rows_per_w)], idxbuf)

        bufs = (xb0, xb1, xb2, xb3)
        ins = (si0, si1, si2, si3)
        outs = (so0, so1, so2, so3)

        def row_chunk(xb, goff):
            bvec = idxbuf[pl.ds(goff, L)]
            for j in range(L // 4):
                bis = [bvec[j + m * (L // 4)] for m in range(4)]

                @plsc.parallel_loop(0, d // L, unroll=8)
                def col_body(k, bis=bis, j=j, xb=xb):
                    for m in range(4):
                        v = wbuf[bis[m], pl.ds(k * L, L)]
                        plsc.addupdate(xb.at[j + m * (L // 4), pl.ds(k * L, L)], v)

        pltpu.async_copy(x_hbm.at[pl.ds(base, R)], bufs[0], ins[0])

        def outer(g4, carry):
            for b in range(4):
                g = g4 * 4 + b
                nb = (b + 1) % 4
                @pl.when(g + 1 < n_chunks)
                def _():
                    @pl.when(g >= 3)
                    def _():
                        pltpu.make_async_copy(
                            bufs[nb], out_hbm.at[pl.ds(base + (g - 3) * R, R)],
                            outs[nb]).wait()

                    pltpu.async_copy(
                        x_hbm.at[pl.ds(base + (g + 1) * R, R)], bufs[nb],
                        ins[nb])

                pltpu.make_async_copy(
                    x_hbm.at[pl.ds(base + g * R, R)], bufs[b], ins[b]).wait()
                row_chunk(bufs[b], g * R)
                pltpu.async_copy(
                    bufs[b], out_hbm.at[pl.ds(base + g * R, R)], outs[b])
            return carry

        lax.fori_loop(0, n_chunks // 4, outer, 0)
        for b in range(4):
            g = n_chunks - 4 + b
            pltpu.make_async_copy(
                bufs[g % 4], out_hbm.at[pl.ds(base + g * R, R)],
                outs[g % 4]).wait()

    out = sc_add(x2, idx, W)
    return out.reshape(B, T, d)
